# Initial kernel scaffold; baseline (speedup 1.0000x reference)
import functools

import jax
import jax.numpy as jnp
from jax import lax
from jax.experimental import pallas as pl
from jax.experimental.pallas import tpu as pltpu
from jax.experimental.pallas import tpu_sc as plsc

N = 10000
IN_DIM = 128
OUT_DIM = 16
E = 320000

NC = 2
NS = 16
L = 16
NW = NC * NS

N_PAD = 10240
RPT = N_PAD // NS
CHUNK = 128
K = 80
EPT = K * CHUNK
E_PAD = EPT * NW

_mesh = plsc.VectorSubcoreMesh(core_axis_name="c", subcore_axis_name="s")


@functools.partial(
    pl.kernel,
    out_type=jax.ShapeDtypeStruct((NC, N_PAD), jnp.float32),
    mesh=_mesh,
    scratch_types=[
        pltpu.VMEM((K, CHUNK), jnp.int32),
        pltpu.VMEM((CHUNK,), jnp.float32),
        pltpu.VMEM((RPT,), jnp.float32),
        pltpu.VMEM_SHARED((N_PAD,), jnp.float32),
    ],
)
def _deg_kernel(dst_hbm, degp_hbm, idx_v, ones_v, zb_v, deg_sh):
    c = lax.axis_index("c")
    s = lax.axis_index("s")
    wid = s * NC + c
    one = jnp.ones((L,), jnp.float32)
    zero = jnp.zeros((L,), jnp.float32)
    for i in range(CHUNK // L):
        ones_v[pl.ds(i * L, L)] = one
    for i in range(RPT // L):
        zb_v[pl.ds(i * L, L)] = zero
    pltpu.sync_copy(zb_v, deg_sh.at[pl.ds(s * RPT, RPT)])
    pltpu.sync_copy(dst_hbm.at[wid], idx_v)
    plsc.subcore_barrier()

    def body(j, carry):
        pltpu.sync_copy(ones_v, deg_sh.at[idx_v.at[j]], add=True)
        return carry

    lax.fori_loop(0, K, body, 0)
    plsc.subcore_barrier()
    pltpu.sync_copy(deg_sh.at[pl.ds(s * RPT, RPT)],
                    degp_hbm.at[c].at[pl.ds(s * RPT, RPT)])


@functools.partial(
    pl.kernel,
    out_type=jax.ShapeDtypeStruct((NC, N_PAD, OUT_DIM), jnp.float32),
    mesh=_mesh,
    scratch_types=[
        pltpu.VMEM((K, CHUNK), jnp.int32),
        pltpu.VMEM((K, CHUNK), jnp.int32),
        pltpu.VMEM((CHUNK, OUT_DIM), jnp.float32),
        pltpu.VMEM_SHARED((N_PAD, OUT_DIM), jnp.float32),
        pltpu.SemaphoreType.DMA,
    ],
)
def _agg_kernel(src_hbm, dst_hbm, g_hbm, accp_hbm, sidx_v, didx_v, rows_v,
                acc_sh, sem):
    c = lax.axis_index("c")
    s = lax.axis_index("s")
    wid = s * NC + c
    zero = jnp.zeros((L,), jnp.float32)
    for i in range(CHUNK):
        rows_v[i, :] = zero
    for t in range(RPT // CHUNK):
        pltpu.sync_copy(rows_v, acc_sh.at[pl.ds(s * RPT + t * CHUNK, CHUNK)])
    pltpu.sync_copy(src_hbm.at[wid], sidx_v)
    pltpu.sync_copy(dst_hbm.at[wid], didx_v)
    plsc.subcore_barrier()

    def body(j, carry):
        pltpu.async_copy(g_hbm.at[sidx_v.at[j]], rows_v, sem).wait()
        pltpu.sync_copy(rows_v, acc_sh.at[didx_v.at[j]], add=True)
        return carry

    lax.fori_loop(0, K, body, 0)
    plsc.subcore_barrier()
    pltpu.sync_copy(acc_sh.at[pl.ds(s * RPT, RPT)],
                    accp_hbm.at[c].at[pl.ds(s * RPT, RPT)])


def _linear_body(x_ref, w_ref, degp_ref, g_ref):
    deg = degp_ref[0, :] + degp_ref[1, :] + 1.0
    dinv = lax.rsqrt(deg)
    h = lax.dot_general(x_ref[...], w_ref[...],
                        (((1,), (1,)), ((), ())),
                        preferred_element_type=jnp.float32)
    g_ref[:N, :] = h * dinv[:N, None]
    g_ref[N:, :] = jnp.zeros((N_PAD - N, OUT_DIM), jnp.float32)


def _combine_body(accp_ref, g_ref, degp_ref, b_ref, out_ref):
    deg = degp_ref[0, :N] + degp_ref[1, :N] + 1.0
    dinv = lax.rsqrt(deg)
    acc = accp_ref[0, :N, :] + accp_ref[1, :N, :] + g_ref[:N, :]
    out_ref[...] = acc * dinv[:, None] + b_ref[...]


def kernel(x, edge_index, W, b):
    src = edge_index[0].astype(jnp.int32)
    dst = edge_index[1].astype(jnp.int32)
    pad = E_PAD - E
    ar = jnp.arange(pad, dtype=jnp.int32)
    src_p = jnp.concatenate([src, ar % 211]).reshape(NW, K, CHUNK)
    dst_p = jnp.concatenate([dst, N + (ar % (N_PAD - N))]).reshape(NW, K, CHUNK)

    degp = _deg_kernel(dst_p)
    g = pl.pallas_call(
        _linear_body,
        out_shape=jax.ShapeDtypeStruct((N_PAD, OUT_DIM), jnp.float32),
    )(x, W, degp)
    accp = _agg_kernel(src_p, dst_p, g)
    out = pl.pallas_call(
        _combine_body,
        out_shape=jax.ShapeDtypeStruct((N, OUT_DIM), jnp.float32),
    )(accp, g, degp, b.reshape(1, OUT_DIM))
    return out


# speedup vs baseline: 50.4902x; 50.4902x over previous
import functools

import jax
import jax.numpy as jnp
from jax import lax
from jax.experimental import pallas as pl
from jax.experimental.pallas import tpu as pltpu
from jax.experimental.pallas import tpu_sc as plsc

N = 10000
IN_DIM = 128
OUT_DIM = 16
E = 320000

NC = 2
NS = 16
L = 16
NW = NC * NS

N_PAD = 10240
RPT = N_PAD // NS
CHUNK = 128
K = 80
EPT = K * CHUNK
E_PAD = EPT * NW

_mesh = plsc.VectorSubcoreMesh(core_axis_name="c", subcore_axis_name="s")


@functools.partial(
    pl.kernel,
    out_type=jax.ShapeDtypeStruct((NC, N_PAD), jnp.float32),
    mesh=_mesh,
    scratch_types=[
        pltpu.VMEM((K, CHUNK), jnp.int32),
        pltpu.VMEM((CHUNK,), jnp.float32),
        pltpu.VMEM((RPT,), jnp.float32),
        pltpu.VMEM_SHARED((N_PAD,), jnp.float32),
    ],
)
def _deg_kernel(dst_hbm, degp_hbm, idx_v, ones_v, zb_v, deg_sh):
    c = lax.axis_index("c")
    s = lax.axis_index("s")
    wid = s * NC + c
    one = jnp.ones((L,), jnp.float32)
    zero = jnp.zeros((L,), jnp.float32)
    for i in range(CHUNK // L):
        ones_v[pl.ds(i * L, L)] = one
    for i in range(RPT // L):
        zb_v[pl.ds(i * L, L)] = zero
    pltpu.sync_copy(zb_v, deg_sh.at[pl.ds(s * RPT, RPT)])
    pltpu.sync_copy(dst_hbm.at[wid], idx_v)
    plsc.subcore_barrier()

    def body(j, carry):
        pltpu.sync_copy(ones_v, deg_sh.at[idx_v.at[j]], add=True)
        return carry

    lax.fori_loop(0, K, body, 0)
    plsc.subcore_barrier()
    pltpu.sync_copy(deg_sh.at[pl.ds(s * RPT, RPT)],
                    degp_hbm.at[c].at[pl.ds(s * RPT, RPT)])


@functools.partial(
    pl.kernel,
    out_type=jax.ShapeDtypeStruct((NC, N_PAD, OUT_DIM), jnp.float32),
    mesh=_mesh,
    scratch_types=[
        pltpu.VMEM((K, CHUNK), jnp.int32),
        pltpu.VMEM((K, CHUNK), jnp.int32),
        pltpu.VMEM((CHUNK, OUT_DIM), jnp.float32),
        pltpu.VMEM_SHARED((N_PAD, OUT_DIM), jnp.float32),
        pltpu.SemaphoreType.DMA,
    ],
    compiler_params=pltpu.CompilerParams(use_tc_tiling_on_sc=False),
)
def _agg_kernel(src_hbm, dst_hbm, g_hbm, accp_hbm, sidx_v, didx_v, rows_v,
                acc_sh, sem):
    c = lax.axis_index("c")
    s = lax.axis_index("s")
    wid = s * NC + c
    zero = jnp.zeros((L,), jnp.float32)
    for i in range(CHUNK):
        rows_v[i, :] = zero
    for t in range(RPT // CHUNK):
        pltpu.sync_copy(rows_v, acc_sh.at[pl.ds(s * RPT + t * CHUNK, CHUNK)])
    pltpu.sync_copy(src_hbm.at[wid], sidx_v)
    pltpu.sync_copy(dst_hbm.at[wid], didx_v)
    plsc.subcore_barrier()

    def body(j, carry):
        pltpu.async_copy(g_hbm.at[sidx_v.at[j]], rows_v, sem).wait()
        pltpu.sync_copy(rows_v, acc_sh.at[didx_v.at[j]], add=True)
        return carry

    lax.fori_loop(0, K, body, 0)
    plsc.subcore_barrier()
    pltpu.sync_copy(acc_sh.at[pl.ds(s * RPT, RPT)],
                    accp_hbm.at[c].at[pl.ds(s * RPT, RPT)])


def _linear_body(x_ref, w_ref, degp_ref, g_ref):
    deg = degp_ref[0, :] + degp_ref[1, :] + 1.0
    dinv = lax.rsqrt(deg)
    h = lax.dot_general(x_ref[...], w_ref[...],
                        (((1,), (1,)), ((), ())),
                        preferred_element_type=jnp.float32)
    g_ref[:N, :] = h * dinv[:N, None]
    g_ref[N:, :] = jnp.zeros((N_PAD - N, OUT_DIM), jnp.float32)


def _combine_body(accp_ref, g_ref, degp_ref, b_ref, out_ref):
    deg = degp_ref[0, :N] + degp_ref[1, :N] + 1.0
    dinv = lax.rsqrt(deg)
    acc = accp_ref[0, :N, :] + accp_ref[1, :N, :] + g_ref[:N, :]
    out_ref[...] = acc * dinv[:, None] + b_ref[...]


def kernel(x, edge_index, W, b):
    src = edge_index[0].astype(jnp.int32)
    dst = edge_index[1].astype(jnp.int32)
    pad = E_PAD - E
    ar = jnp.arange(pad, dtype=jnp.int32)
    src_p = jnp.concatenate([src, ar % 211]).reshape(NW, K, CHUNK)
    dst_p = jnp.concatenate([dst, N + (ar % (N_PAD - N))]).reshape(NW, K, CHUNK)

    degp = _deg_kernel(dst_p)
    g = pl.pallas_call(
        _linear_body,
        out_shape=jax.ShapeDtypeStruct((N_PAD, OUT_DIM), jnp.float32),
    )(x, W, degp)
    accp = _agg_kernel(src_p, dst_p, g)
    out = pl.pallas_call(
        _combine_body,
        out_shape=jax.ShapeDtypeStruct((N, OUT_DIM), jnp.float32),
    )(accp, g, degp, b.reshape(1, OUT_DIM))
    return out


# speedup vs baseline: 83.1288x; 1.6464x over previous
import functools

import jax
import jax.numpy as jnp
from jax import lax
from jax.experimental import pallas as pl
from jax.experimental.pallas import tpu as pltpu
from jax.experimental.pallas import tpu_sc as plsc

N = 10000
IN_DIM = 128
OUT_DIM = 16
E = 320000

NC = 2
NS = 16
L = 16
NW = NC * NS

N_PAD = 10240
RPT = N_PAD // NS
CHUNK = 128
K = 80
EPT = K * CHUNK
E_PAD = EPT * NW

_mesh = plsc.VectorSubcoreMesh(core_axis_name="c", subcore_axis_name="s")

NB = 8
KB = K // NB


@functools.partial(
    pl.kernel,
    out_type=jax.ShapeDtypeStruct((NC, N_PAD), jnp.float32),
    mesh=_mesh,
    scratch_types=[
        pltpu.VMEM((K, CHUNK), jnp.int32),
        pltpu.VMEM((CHUNK,), jnp.float32),
        pltpu.VMEM((RPT,), jnp.float32),
        pltpu.VMEM_SHARED((N_PAD,), jnp.float32),
    ] + [pltpu.SemaphoreType.DMA] * NB,
)
def _deg_kernel(dst_hbm, degp_hbm, idx_v, ones_v, zb_v, deg_sh, *ssem):
    c = lax.axis_index("c")
    s = lax.axis_index("s")
    wid = s * NC + c
    one = jnp.ones((L,), jnp.float32)
    zero = jnp.zeros((L,), jnp.float32)
    for i in range(CHUNK // L):
        ones_v[pl.ds(i * L, L)] = one
    for i in range(RPT // L):
        zb_v[pl.ds(i * L, L)] = zero
    pltpu.sync_copy(zb_v, deg_sh.at[pl.ds(s * RPT, RPT)])
    pltpu.sync_copy(dst_hbm.at[wid], idx_v)
    plsc.subcore_barrier()

    for b in range(NB):
        pltpu.async_copy(ones_v, deg_sh.at[idx_v.at[b]], ssem[b], add=True)

    def body(t, carry):
        for b in range(NB):
            j = t * NB + b
            pltpu.make_async_copy(ones_v, deg_sh.at[idx_v.at[j]],
                                  ssem[b]).wait()
            pltpu.async_copy(ones_v, deg_sh.at[idx_v.at[j + NB]], ssem[b],
                             add=True)
        return carry

    lax.fori_loop(0, KB - 1, body, 0)
    for b in range(NB):
        j = (KB - 1) * NB + b
        pltpu.make_async_copy(ones_v, deg_sh.at[idx_v.at[j]], ssem[b]).wait()
    plsc.subcore_barrier()
    pltpu.sync_copy(deg_sh.at[pl.ds(s * RPT, RPT)],
                    degp_hbm.at[c].at[pl.ds(s * RPT, RPT)])


@functools.partial(
    pl.kernel,
    out_type=jax.ShapeDtypeStruct((NC, N_PAD, OUT_DIM), jnp.float32),
    mesh=_mesh,
    scratch_types=[
        pltpu.VMEM((K, CHUNK), jnp.int32),
        pltpu.VMEM((K, CHUNK), jnp.int32),
        pltpu.VMEM((NB, CHUNK, OUT_DIM), jnp.float32),
        pltpu.VMEM_SHARED((N_PAD, OUT_DIM), jnp.float32),
    ] + [pltpu.SemaphoreType.DMA] * (2 * NB),
    compiler_params=pltpu.CompilerParams(use_tc_tiling_on_sc=False),
)
def _agg_kernel(src_hbm, dst_hbm, g_hbm, accp_hbm, sidx_v, didx_v, rows_v,
                acc_sh, *sems):
    gsem, ssem = sems[:NB], sems[NB:]
    c = lax.axis_index("c")
    s = lax.axis_index("s")
    wid = s * NC + c
    zero = jnp.zeros((L,), jnp.float32)
    for i in range(CHUNK):
        rows_v[0, i, :] = zero
    for t in range(RPT // CHUNK):
        pltpu.sync_copy(rows_v.at[0],
                        acc_sh.at[pl.ds(s * RPT + t * CHUNK, CHUNK)])
    pltpu.sync_copy(src_hbm.at[wid], sidx_v)
    pltpu.sync_copy(dst_hbm.at[wid], didx_v)
    plsc.subcore_barrier()

    for b in range(NB):
        pltpu.async_copy(g_hbm.at[sidx_v.at[b]], rows_v.at[b], gsem[b])

    def body(t, carry):
        j0 = t * NB
        for b in range(NB):
            j = j0 + b
            pltpu.make_async_copy(g_hbm.at[sidx_v.at[j]], rows_v.at[b],
                                  gsem[b]).wait()
            pltpu.async_copy(rows_v.at[b], acc_sh.at[didx_v.at[j]], ssem[b],
                             add=True)
        for b in range(NB):
            j = j0 + b
            pltpu.make_async_copy(rows_v.at[b], acc_sh.at[didx_v.at[j]],
                                  ssem[b]).wait()
            pltpu.async_copy(g_hbm.at[sidx_v.at[j + NB]], rows_v.at[b],
                             gsem[b])
        return carry

    lax.fori_loop(0, KB - 1, body, 0)
    for b in range(NB):
        j = (KB - 1) * NB + b
        pltpu.make_async_copy(g_hbm.at[sidx_v.at[j]], rows_v.at[b],
                              gsem[b]).wait()
        pltpu.async_copy(rows_v.at[b], acc_sh.at[didx_v.at[j]], ssem[b],
                         add=True)
    for b in range(NB):
        j = (KB - 1) * NB + b
        pltpu.make_async_copy(rows_v.at[b], acc_sh.at[didx_v.at[j]],
                              ssem[b]).wait()
    plsc.subcore_barrier()
    pltpu.sync_copy(acc_sh.at[pl.ds(s * RPT, RPT)],
                    accp_hbm.at[c].at[pl.ds(s * RPT, RPT)])


def _linear_body(x_ref, w_ref, degp_ref, g_ref):
    deg = degp_ref[0, :] + degp_ref[1, :] + 1.0
    dinv = lax.rsqrt(deg)
    h = lax.dot_general(x_ref[...], w_ref[...],
                        (((1,), (1,)), ((), ())),
                        preferred_element_type=jnp.float32)
    g_ref[:N, :] = h * dinv[:N, None]
    g_ref[N:, :] = jnp.zeros((N_PAD - N, OUT_DIM), jnp.float32)


def _combine_body(accp_ref, g_ref, degp_ref, b_ref, out_ref):
    deg = degp_ref[0, :N] + degp_ref[1, :N] + 1.0
    dinv = lax.rsqrt(deg)
    acc = accp_ref[0, :N, :] + accp_ref[1, :N, :] + g_ref[:N, :]
    out_ref[...] = acc * dinv[:, None] + b_ref[...]


def kernel(x, edge_index, W, b):
    src = edge_index[0].astype(jnp.int32)
    dst = edge_index[1].astype(jnp.int32)
    pad = E_PAD - E
    ar = jnp.arange(pad, dtype=jnp.int32)
    src_p = jnp.concatenate([src, ar % 211]).reshape(NW, K, CHUNK)
    dst_p = jnp.concatenate([dst, N + (ar % (N_PAD - N))]).reshape(NW, K, CHUNK)

    degp = _deg_kernel(dst_p)
    g = pl.pallas_call(
        _linear_body,
        out_shape=jax.ShapeDtypeStruct((N_PAD, OUT_DIM), jnp.float32),
    )(x, W, degp)
    accp = _agg_kernel(src_p, dst_p, g)
    out = pl.pallas_call(
        _combine_body,
        out_shape=jax.ShapeDtypeStruct((N, OUT_DIM), jnp.float32),
    )(accp, g, degp, b.reshape(1, OUT_DIM))
    return out
